# Initial kernel scaffold; baseline (speedup 1.0000x reference)
import functools

import jax
import jax.numpy as jnp
from jax import lax
from jax.experimental import pallas as pl
from jax.experimental.pallas import tpu as pltpu
from jax.experimental.pallas import tpu_sc as plsc

N_NODES = 100000
N_EDGES = 6400000
NP = 102400
CH = 128
GRP = 16
N_OUTER = N_EDGES // (CH * GRP)
NC = 2
NS = 16
NW = NC * NS
OUTER_LO = N_OUTER // NW
OUTER_EXTRA = N_OUTER % NW
NPT = NP // NS

_mesh = plsc.VectorSubcoreMesh(
    core_axis_name="c", subcore_axis_name="s", num_cores=NC, num_subcores=NS
)


def _worker_range(c, s):
    wid = s * NC + c
    base = wid * OUTER_LO + jnp.minimum(wid, OUTER_EXTRA)
    trips = OUTER_LO + jnp.where(wid < OUTER_EXTRA, 1, 0)
    return base, trips


@functools.partial(
    pl.kernel,
    out_type=jax.ShapeDtypeStruct((NC, NP), jnp.float32),
    mesh=_mesh,
    scratch_types=[
        pltpu.VMEM((GRP, CH), jnp.int32),
        pltpu.VMEM((GRP, CH), jnp.float32),
        pltpu.VMEM((NPT,), jnp.float32),
        pltpu.VMEM_SHARED((NP,), jnp.float32),
    ],
)
def _deg_kernel(col2d, w2d, degp, cbuf, wbuf, zbuf, deg_sh):
    c = lax.axis_index("c")
    s = lax.axis_index("s")

    def _z(i, _):
        zbuf[pl.ds(i * 16, 16)] = jnp.zeros((16,), jnp.float32)
        return 0
    lax.fori_loop(0, NPT // 16, _z, 0)
    pltpu.sync_copy(zbuf, deg_sh.at[pl.ds(s * NPT, NPT)])
    plsc.subcore_barrier()

    base, trips = _worker_range(c, s)

    def _outer(o, _):
        g = base + o
        pltpu.sync_copy(col2d.at[pl.ds(g * GRP, GRP)], cbuf)
        pltpu.sync_copy(w2d.at[pl.ds(g * GRP, GRP)], wbuf)
        for j in range(GRP):
            pltpu.sync_copy(wbuf.at[j], deg_sh.at[cbuf.at[j]], add=True)
        return 0
    lax.fori_loop(0, trips, _outer, 0)

    plsc.subcore_barrier()
    pltpu.sync_copy(deg_sh.at[pl.ds(s * NPT, NPT)],
                    degp.at[c, pl.ds(s * NPT, NPT)])


@functools.partial(
    pl.kernel,
    out_type=jax.ShapeDtypeStruct((NC, NP, 8), jnp.float32),
    mesh=_mesh,
    scratch_types=[
        pltpu.VMEM((GRP, CH), jnp.int32),
        pltpu.VMEM((GRP, CH), jnp.int32),
        pltpu.VMEM((GRP, CH), jnp.float32),
        pltpu.VMEM((CH, 8), jnp.float32),
        pltpu.VMEM((NPT, 8), jnp.float32),
        pltpu.VMEM_SHARED((NP, 8), jnp.float32),
        pltpu.SemaphoreType.DMA,
    ],
)
def _agg_kernel(row2d, col2d, w2d, xs, z8, aggp,
                rbuf, cbuf, wbuf, rows, zbuf, agg_sh, sem):
    c = lax.axis_index("c")
    s = lax.axis_index("s")

    pltpu.sync_copy(z8.at[pl.ds(s * NPT, NPT)], zbuf)
    pltpu.sync_copy(zbuf, agg_sh.at[pl.ds(s * NPT, NPT)])
    plsc.subcore_barrier()

    base, trips = _worker_range(c, s)

    iota = lax.iota(jnp.int32, 16)
    ge8 = jnp.where(iota >= 8, 1, 0)
    mod8 = iota - 8 * ge8

    def _outer(o, _):
        g = base + o
        pltpu.sync_copy(row2d.at[pl.ds(g * GRP, GRP)], rbuf)
        pltpu.sync_copy(col2d.at[pl.ds(g * GRP, GRP)], cbuf)
        pltpu.sync_copy(w2d.at[pl.ds(g * GRP, GRP)], wbuf)
        for j in range(GRP):
            pltpu.async_copy(xs.at[rbuf.at[j]], rows, sem).wait()
            wrow = wbuf.at[j]

            def _scale(v, _):
                ridx = 2 * v + ge8
                wv = plsc.load_gather(wrow, [ridx])
                rv = plsc.load_gather(rows, [ridx, mod8])
                plsc.store_scatter(rows, [ridx, mod8], wv * rv)
                return 0
            lax.fori_loop(0, CH // 2, _scale, 0)
            pltpu.sync_copy(rows, agg_sh.at[cbuf.at[j]], add=True)
        return 0
    lax.fori_loop(0, trips, _outer, 0)

    plsc.subcore_barrier()
    pltpu.sync_copy(agg_sh.at[pl.ds(s * NPT, NPT)],
                    aggp.at[c, pl.ds(s * NPT, NPT)])


def _t1_body(degp_ref, x_ref, dinv_ref, xs_ref):
    deg = 1.0 + degp_ref[0, :] + degp_ref[1, :]
    dinv = lax.rsqrt(deg)
    dinv_ref[...] = dinv[:, None]
    xs_ref[...] = dinv[:, None] * x_ref[...]


def _t2_body(aggp_ref, xs_ref, dinv_ref, w_ref, b_ref, out_ref):
    t = (aggp_ref[0] + aggp_ref[1] + xs_ref[...]) * dinv_ref[...]
    out_ref[...] = (
        jnp.dot(t, w_ref[...], preferred_element_type=jnp.float32) + b_ref[...]
    )


_BT = 2048


def kernel(x, edge_index, edge_attr, W, b):
    row = edge_index[0].astype(jnp.int32)
    col = edge_index[1].astype(jnp.int32)
    w = edge_attr.astype(jnp.float32)
    ech = N_EDGES // CH
    row2d = row.reshape(ech, CH)
    col2d = col.reshape(ech, CH)
    w2d = w.reshape(ech, CH)
    x_p = jnp.pad(x, ((0, NP - N_NODES), (0, 0)))
    z8 = jnp.zeros((NP, 8), jnp.float32)

    degp = _deg_kernel(col2d, w2d)

    dinv, xs = pl.pallas_call(
        _t1_body,
        grid=(NP // _BT,),
        in_specs=[
            pl.BlockSpec((NC, _BT), lambda i: (0, i)),
            pl.BlockSpec((_BT, 8), lambda i: (i, 0)),
        ],
        out_specs=[
            pl.BlockSpec((_BT, 1), lambda i: (i, 0)),
            pl.BlockSpec((_BT, 8), lambda i: (i, 0)),
        ],
        out_shape=[
            jax.ShapeDtypeStruct((NP, 1), jnp.float32),
            jax.ShapeDtypeStruct((NP, 8), jnp.float32),
        ],
    )(degp, x_p)

    aggp = _agg_kernel(row2d, col2d, w2d, xs, z8)

    out = pl.pallas_call(
        _t2_body,
        grid=(NP // _BT,),
        in_specs=[
            pl.BlockSpec((NC, _BT, 8), lambda i: (0, i, 0)),
            pl.BlockSpec((_BT, 8), lambda i: (i, 0)),
            pl.BlockSpec((_BT, 1), lambda i: (i, 0)),
            pl.BlockSpec((8, 64), lambda i: (0, 0)),
            pl.BlockSpec((1, 64), lambda i: (0, 0)),
        ],
        out_specs=pl.BlockSpec((_BT, 64), lambda i: (i, 0)),
        out_shape=jax.ShapeDtypeStruct((NP, 64), jnp.float32),
    )(aggp, xs, dinv, W, b.reshape(1, 64))

    return out[:N_NODES]


# speedup vs baseline: 49.8071x; 49.8071x over previous
import functools

import jax
import jax.numpy as jnp
from jax import lax
from jax.experimental import pallas as pl
from jax.experimental.pallas import tpu as pltpu
from jax.experimental.pallas import tpu_sc as plsc

N_NODES = 100000
N_EDGES = 6400000
NP = 102400
CH = 128
GRP = 16
N_OUTER = N_EDGES // (CH * GRP)
NC = 2
NS = 16
NW = NC * NS
OUTER_LO = N_OUTER // NW
OUTER_EXTRA = N_OUTER % NW
NPT = NP // NS

_mesh = plsc.VectorSubcoreMesh(
    core_axis_name="c", subcore_axis_name="s", num_cores=NC, num_subcores=NS
)

_SC_PARAMS = pltpu.CompilerParams(
    needs_layout_passes=False, use_tc_tiling_on_sc=False
)


def _worker_range(c, s):
    wid = s * NC + c
    base = wid * OUTER_LO + jnp.minimum(wid, OUTER_EXTRA)
    trips = OUTER_LO + jnp.where(wid < OUTER_EXTRA, 1, 0)
    return base, trips


@functools.partial(
    pl.kernel,
    out_type=jax.ShapeDtypeStruct((NC, NP), jnp.float32),
    mesh=_mesh,
    compiler_params=_SC_PARAMS,
    scratch_types=[
        pltpu.VMEM((GRP, CH), jnp.int32),
        pltpu.VMEM((GRP, CH), jnp.float32),
        pltpu.VMEM((NPT,), jnp.float32),
        pltpu.VMEM_SHARED((NP,), jnp.float32),
    ],
)
def _deg_kernel(col2d, w2d, degp, cbuf, wbuf, zbuf, deg_sh):
    c = lax.axis_index("c")
    s = lax.axis_index("s")

    def _z(i, _):
        zbuf[pl.ds(i * 16, 16)] = jnp.zeros((16,), jnp.float32)
        return 0
    lax.fori_loop(0, NPT // 16, _z, 0)
    pltpu.sync_copy(zbuf, deg_sh.at[pl.ds(s * NPT, NPT)])
    plsc.subcore_barrier()

    base, trips = _worker_range(c, s)

    def _outer(o, _):
        g = base + o
        pltpu.sync_copy(col2d.at[pl.ds(g * GRP, GRP)], cbuf)
        pltpu.sync_copy(w2d.at[pl.ds(g * GRP, GRP)], wbuf)
        for j in range(GRP):
            pltpu.sync_copy(wbuf.at[j], deg_sh.at[cbuf.at[j]], add=True)
        return 0
    lax.fori_loop(0, trips, _outer, 0)

    plsc.subcore_barrier()
    pltpu.sync_copy(deg_sh.at[pl.ds(s * NPT, NPT)],
                    degp.at[c, pl.ds(s * NPT, NPT)])


@functools.partial(
    pl.kernel,
    out_type=jax.ShapeDtypeStruct((NC, NP, 8), jnp.float32),
    mesh=_mesh,
    compiler_params=_SC_PARAMS,
    scratch_types=[
        pltpu.VMEM((GRP, CH), jnp.int32),
        pltpu.VMEM((GRP, CH), jnp.int32),
        pltpu.VMEM((GRP, CH), jnp.float32),
        pltpu.VMEM((CH, 8), jnp.float32),
        pltpu.VMEM((NPT, 8), jnp.float32),
        pltpu.VMEM_SHARED((NP, 8), jnp.float32),
        pltpu.SemaphoreType.DMA,
    ],
)
def _agg_kernel(row2d, col2d, w2d, xs, z8, aggp,
                rbuf, cbuf, wbuf, rows, zbuf, agg_sh, sem):
    c = lax.axis_index("c")
    s = lax.axis_index("s")

    pltpu.sync_copy(z8.at[pl.ds(s * NPT, NPT)], zbuf)
    pltpu.sync_copy(zbuf, agg_sh.at[pl.ds(s * NPT, NPT)])
    plsc.subcore_barrier()

    base, trips = _worker_range(c, s)

    iota = lax.iota(jnp.int32, 16)
    ge8 = jnp.where(iota >= 8, 1, 0)
    mod8 = iota - 8 * ge8

    def _outer(o, _):
        g = base + o
        pltpu.sync_copy(row2d.at[pl.ds(g * GRP, GRP)], rbuf)
        pltpu.sync_copy(col2d.at[pl.ds(g * GRP, GRP)], cbuf)
        pltpu.sync_copy(w2d.at[pl.ds(g * GRP, GRP)], wbuf)
        for j in range(GRP):
            pltpu.async_copy(xs.at[rbuf.at[j]], rows, sem).wait()
            jvec = jnp.full((16,), j, jnp.int32)

            def _scale(v, _):
                ridx = 2 * v + ge8
                wv = plsc.load_gather(wbuf, [jvec, ridx])
                rv = plsc.load_gather(rows, [ridx, mod8])
                plsc.store_scatter(rows, [ridx, mod8], wv * rv)
                return 0
            lax.fori_loop(0, CH // 2, _scale, 0)
            pltpu.sync_copy(rows, agg_sh.at[cbuf.at[j]], add=True)
        return 0
    lax.fori_loop(0, trips, _outer, 0)

    plsc.subcore_barrier()
    pltpu.sync_copy(agg_sh.at[pl.ds(s * NPT, NPT)],
                    aggp.at[c, pl.ds(s * NPT, NPT)])


def _t1_body(degp_ref, x_ref, dinv_ref, xs_ref):
    deg = 1.0 + degp_ref[0, :] + degp_ref[1, :]
    dinv = lax.rsqrt(deg)
    dinv_ref[...] = dinv[:, None]
    xs_ref[...] = dinv[:, None] * x_ref[...]


def _t2_body(aggp_ref, xs_ref, dinv_ref, w_ref, b_ref, out_ref):
    t = (aggp_ref[0] + aggp_ref[1] + xs_ref[...]) * dinv_ref[...]
    out_ref[...] = (
        jnp.dot(t, w_ref[...], preferred_element_type=jnp.float32) + b_ref[...]
    )


_BT = 2048


def kernel(x, edge_index, edge_attr, W, b):
    row = edge_index[0].astype(jnp.int32)
    col = edge_index[1].astype(jnp.int32)
    w = edge_attr.astype(jnp.float32)
    ech = N_EDGES // CH
    row2d = row.reshape(ech, CH)
    col2d = col.reshape(ech, CH)
    w2d = w.reshape(ech, CH)
    x_p = jnp.pad(x, ((0, NP - N_NODES), (0, 0)))
    z8 = jnp.zeros((NP, 8), jnp.float32)

    degp = _deg_kernel(col2d, w2d)

    dinv, xs = pl.pallas_call(
        _t1_body,
        grid=(NP // _BT,),
        in_specs=[
            pl.BlockSpec((NC, _BT), lambda i: (0, i)),
            pl.BlockSpec((_BT, 8), lambda i: (i, 0)),
        ],
        out_specs=[
            pl.BlockSpec((_BT, 1), lambda i: (i, 0)),
            pl.BlockSpec((_BT, 8), lambda i: (i, 0)),
        ],
        out_shape=[
            jax.ShapeDtypeStruct((NP, 1), jnp.float32),
            jax.ShapeDtypeStruct((NP, 8), jnp.float32),
        ],
    )(degp, x_p)

    aggp = _agg_kernel(row2d, col2d, w2d, xs, z8)

    out = pl.pallas_call(
        _t2_body,
        grid=(NP // _BT,),
        in_specs=[
            pl.BlockSpec((NC, _BT, 8), lambda i: (0, i, 0)),
            pl.BlockSpec((_BT, 8), lambda i: (i, 0)),
            pl.BlockSpec((_BT, 1), lambda i: (i, 0)),
            pl.BlockSpec((8, 64), lambda i: (0, 0)),
            pl.BlockSpec((1, 64), lambda i: (0, 0)),
        ],
        out_specs=pl.BlockSpec((_BT, 64), lambda i: (i, 0)),
        out_shape=jax.ShapeDtypeStruct((NP, 64), jnp.float32),
    )(aggp, xs, dinv, W, b.reshape(1, 64))

    return out[:N_NODES]


# speedup vs baseline: 65.4188x; 1.3134x over previous
import functools

import jax
import jax.numpy as jnp
from jax import lax
from jax.experimental import pallas as pl
from jax.experimental.pallas import tpu as pltpu
from jax.experimental.pallas import tpu_sc as plsc

N_NODES = 100000
N_EDGES = 6400000
NP = 102400
CH = 128
GRP = 16
N_OUTER = N_EDGES // (CH * GRP)
NC = 2
NS = 16
NW = NC * NS
OUTER_LO = N_OUTER // NW
OUTER_EXTRA = N_OUTER % NW
NPT = NP // NS

_mesh = plsc.VectorSubcoreMesh(
    core_axis_name="c", subcore_axis_name="s", num_cores=NC, num_subcores=NS
)

_SC_PARAMS = pltpu.CompilerParams(
    needs_layout_passes=False, use_tc_tiling_on_sc=False
)


def _worker_range(c, s):
    wid = s * NC + c
    base = wid * OUTER_LO + jnp.minimum(wid, OUTER_EXTRA)
    trips = OUTER_LO + jnp.where(wid < OUTER_EXTRA, 1, 0)
    return base, trips


@functools.partial(
    pl.kernel,
    out_type=jax.ShapeDtypeStruct((NC, NP), jnp.float32),
    mesh=_mesh,
    compiler_params=_SC_PARAMS,
    scratch_types=[
        pltpu.VMEM((GRP, CH), jnp.int32),
        pltpu.VMEM((GRP, CH), jnp.float32),
        pltpu.VMEM((NPT,), jnp.float32),
        pltpu.VMEM_SHARED((NP,), jnp.float32),
    ],
)
def _deg_kernel(col2d, w2d, degp, cbuf, wbuf, zbuf, deg_sh):
    c = lax.axis_index("c")
    s = lax.axis_index("s")

    def _z(i, _):
        zbuf[pl.ds(i * 16, 16)] = jnp.zeros((16,), jnp.float32)
        return 0
    lax.fori_loop(0, NPT // 16, _z, 0)
    pltpu.sync_copy(zbuf, deg_sh.at[pl.ds(s * NPT, NPT)])
    plsc.subcore_barrier()

    base, trips = _worker_range(c, s)

    def _outer(o, _):
        g = base + o
        pltpu.sync_copy(col2d.at[pl.ds(g * GRP, GRP)], cbuf)
        pltpu.sync_copy(w2d.at[pl.ds(g * GRP, GRP)], wbuf)
        for j in range(GRP):
            pltpu.sync_copy(wbuf.at[j], deg_sh.at[cbuf.at[j]], add=True)
        return 0
    lax.fori_loop(0, trips, _outer, 0)

    plsc.subcore_barrier()
    pltpu.sync_copy(deg_sh.at[pl.ds(s * NPT, NPT)],
                    degp.at[c, pl.ds(s * NPT, NPT)])


@functools.partial(
    pl.kernel,
    out_type=jax.ShapeDtypeStruct((NC, NP, 8), jnp.float32),
    mesh=_mesh,
    compiler_params=_SC_PARAMS,
    scratch_types=[
        pltpu.VMEM((GRP, CH), jnp.int32),
        pltpu.VMEM((GRP, CH), jnp.int32),
        pltpu.VMEM((GRP, CH), jnp.float32),
        pltpu.VMEM((2, CH, 8), jnp.float32),
        pltpu.VMEM((NPT, 8), jnp.float32),
        pltpu.VMEM_SHARED((NP, 8), jnp.float32),
        pltpu.SemaphoreType.DMA,
        pltpu.SemaphoreType.DMA,
    ],
)
def _agg_kernel(row2d, col2d, w2d, xs, z8, aggp,
                rbuf, cbuf, wbuf, rows2, zbuf, agg_sh, sg0, sg1):
    c = lax.axis_index("c")
    s = lax.axis_index("s")

    pltpu.sync_copy(z8.at[pl.ds(s * NPT, NPT)], zbuf)
    pltpu.sync_copy(zbuf, agg_sh.at[pl.ds(s * NPT, NPT)])
    plsc.subcore_barrier()

    base, trips = _worker_range(c, s)

    iota = lax.iota(jnp.int32, 16)
    ge8 = jnp.where(iota >= 8, 1, 0)
    mod8 = iota - 8 * ge8
    sems = (sg0, sg1)

    def _outer(o, _):
        g = base + o
        pltpu.sync_copy(row2d.at[pl.ds(g * GRP, GRP)], rbuf)
        pltpu.sync_copy(col2d.at[pl.ds(g * GRP, GRP)], cbuf)
        pltpu.sync_copy(w2d.at[pl.ds(g * GRP, GRP)], wbuf)
        cps = [pltpu.async_copy(xs.at[rbuf.at[0]], rows2.at[0], sems[0]), None]
        for j in range(GRP):
            cur = j & 1
            if j + 1 < GRP:
                cps[1 - cur] = pltpu.async_copy(
                    xs.at[rbuf.at[j + 1]], rows2.at[1 - cur], sems[1 - cur])
            cps[cur].wait()
            rows = rows2.at[cur]
            jvec = jnp.full((16,), j, jnp.int32)
            for v in range(CH // 2):
                ridx = 2 * v + ge8
                wv = plsc.load_gather(wbuf, [jvec, ridx])
                rv = plsc.load_gather(rows, [ridx, mod8])
                plsc.store_scatter(rows, [ridx, mod8], wv * rv)
            pltpu.sync_copy(rows, agg_sh.at[cbuf.at[j]], add=True)
        return 0
    lax.fori_loop(0, trips, _outer, 0)

    plsc.subcore_barrier()
    pltpu.sync_copy(agg_sh.at[pl.ds(s * NPT, NPT)],
                    aggp.at[c, pl.ds(s * NPT, NPT)])


def _t1_body(degp_ref, x_ref, dinv_ref, xs_ref):
    deg = 1.0 + degp_ref[0, :] + degp_ref[1, :]
    dinv = lax.rsqrt(deg)
    dinv_ref[...] = dinv[:, None]
    xs_ref[...] = dinv[:, None] * x_ref[...]


def _t2_body(aggp_ref, xs_ref, dinv_ref, w_ref, b_ref, out_ref):
    t = (aggp_ref[0] + aggp_ref[1] + xs_ref[...]) * dinv_ref[...]
    out_ref[...] = (
        jnp.dot(t, w_ref[...], preferred_element_type=jnp.float32) + b_ref[...]
    )


_BT = 2048


def kernel(x, edge_index, edge_attr, W, b):
    row = edge_index[0].astype(jnp.int32)
    col = edge_index[1].astype(jnp.int32)
    w = edge_attr.astype(jnp.float32)
    ech = N_EDGES // CH
    row2d = row.reshape(ech, CH)
    col2d = col.reshape(ech, CH)
    w2d = w.reshape(ech, CH)
    x_p = jnp.pad(x, ((0, NP - N_NODES), (0, 0)))
    z8 = jnp.zeros((NP, 8), jnp.float32)

    degp = _deg_kernel(col2d, w2d)

    dinv, xs = pl.pallas_call(
        _t1_body,
        grid=(NP // _BT,),
        in_specs=[
            pl.BlockSpec((NC, _BT), lambda i: (0, i)),
            pl.BlockSpec((_BT, 8), lambda i: (i, 0)),
        ],
        out_specs=[
            pl.BlockSpec((_BT, 1), lambda i: (i, 0)),
            pl.BlockSpec((_BT, 8), lambda i: (i, 0)),
        ],
        out_shape=[
            jax.ShapeDtypeStruct((NP, 1), jnp.float32),
            jax.ShapeDtypeStruct((NP, 8), jnp.float32),
        ],
    )(degp, x_p)

    aggp = _agg_kernel(row2d, col2d, w2d, xs, z8)

    out = pl.pallas_call(
        _t2_body,
        grid=(NP // _BT,),
        in_specs=[
            pl.BlockSpec((NC, _BT, 8), lambda i: (0, i, 0)),
            pl.BlockSpec((_BT, 8), lambda i: (i, 0)),
            pl.BlockSpec((_BT, 1), lambda i: (i, 0)),
            pl.BlockSpec((8, 64), lambda i: (0, 0)),
            pl.BlockSpec((1, 64), lambda i: (0, 0)),
        ],
        out_specs=pl.BlockSpec((_BT, 64), lambda i: (i, 0)),
        out_shape=jax.ShapeDtypeStruct((NP, 64), jnp.float32),
    )(aggp, xs, dinv, W, b.reshape(1, 64))

    return out[:N_NODES]


# speedup vs baseline: 74.5631x; 1.1398x over previous
import functools

import jax
import jax.numpy as jnp
from jax import lax
from jax.experimental import pallas as pl
from jax.experimental.pallas import tpu as pltpu
from jax.experimental.pallas import tpu_sc as plsc

N_NODES = 100000
N_EDGES = 6400000
NP = 102400
CH = 128
GRP = 16
N_OUTER = N_EDGES // (CH * GRP)
NC = 2
NS = 16
NW = NC * NS
OUTER_LO = N_OUTER // NW
OUTER_EXTRA = N_OUTER % NW
NPT = NP // NS

_mesh = plsc.VectorSubcoreMesh(
    core_axis_name="c", subcore_axis_name="s", num_cores=NC, num_subcores=NS
)

_SC_PARAMS = pltpu.CompilerParams(
    needs_layout_passes=False, use_tc_tiling_on_sc=False
)


def _worker_range(c, s):
    wid = s * NC + c
    base = wid * OUTER_LO + jnp.minimum(wid, OUTER_EXTRA)
    trips = OUTER_LO + jnp.where(wid < OUTER_EXTRA, 1, 0)
    return base, trips


@functools.partial(
    pl.kernel,
    out_type=jax.ShapeDtypeStruct((NC, NP), jnp.float32),
    mesh=_mesh,
    compiler_params=_SC_PARAMS,
    scratch_types=[
        pltpu.VMEM((GRP, CH), jnp.int32),
        pltpu.VMEM((GRP, CH), jnp.float32),
        pltpu.VMEM((NPT,), jnp.float32),
        pltpu.VMEM_SHARED((NP,), jnp.float32),
        pltpu.SemaphoreType.DMA,
    ],
)
def _deg_kernel(col2d, w2d, degp, cbuf, wbuf, zbuf, deg_sh, sem):
    c = lax.axis_index("c")
    s = lax.axis_index("s")

    def _z(i, _):
        zbuf[pl.ds(i * 16, 16)] = jnp.zeros((16,), jnp.float32)
        return 0
    lax.fori_loop(0, NPT // 16, _z, 0)
    pltpu.sync_copy(zbuf, deg_sh.at[pl.ds(s * NPT, NPT)])
    plsc.subcore_barrier()

    base, trips = _worker_range(c, s)

    def _outer(o, _):
        g = base + o
        pltpu.sync_copy(col2d.at[pl.ds(g * GRP, GRP)], cbuf)
        pltpu.sync_copy(w2d.at[pl.ds(g * GRP, GRP)], wbuf)
        cps = [
            pltpu.async_copy(wbuf.at[j], deg_sh.at[cbuf.at[j]], sem, add=True)
            for j in range(GRP)
        ]
        for cp in cps:
            cp.wait()
        return 0
    lax.fori_loop(0, trips, _outer, 0)

    plsc.subcore_barrier()
    pltpu.sync_copy(deg_sh.at[pl.ds(s * NPT, NPT)],
                    degp.at[c, pl.ds(s * NPT, NPT)])


@functools.partial(
    pl.kernel,
    out_type=jax.ShapeDtypeStruct((NC, NP, 8), jnp.float32),
    mesh=_mesh,
    compiler_params=_SC_PARAMS,
    scratch_types=[
        pltpu.VMEM((GRP, CH), jnp.int32),
        pltpu.VMEM((GRP, CH), jnp.int32),
        pltpu.VMEM((GRP, CH), jnp.float32),
        pltpu.VMEM((2, CH, 8), jnp.float32),
        pltpu.VMEM_SHARED((NP, 8), jnp.float32),
        pltpu.VMEM_SHARED((NP, 8), jnp.float32),
        pltpu.SemaphoreType.DMA,
        pltpu.SemaphoreType.DMA,
        pltpu.SemaphoreType.DMA,
        pltpu.SemaphoreType.DMA,
    ],
)
def _agg_kernel(row2d, col2d, w2d, xs, z8, aggp,
                rbuf, cbuf, wbuf, rows2, agg_sh, xs_sh, sg0, sg1, ss0, ss1):
    c = lax.axis_index("c")
    s = lax.axis_index("s")

    pltpu.sync_copy(z8.at[pl.ds(s * NPT, NPT)], agg_sh.at[pl.ds(s * NPT, NPT)])
    pltpu.sync_copy(xs.at[pl.ds(s * NPT, NPT)], xs_sh.at[pl.ds(s * NPT, NPT)])
    plsc.subcore_barrier()

    base, trips = _worker_range(c, s)

    iota = lax.iota(jnp.int32, 16)
    ge8 = jnp.where(iota >= 8, 1, 0)
    mod8 = iota - 8 * ge8
    gsems = (sg0, sg1)
    ssems = (ss0, ss1)

    def _outer(o, _):
        g = base + o
        pltpu.sync_copy(row2d.at[pl.ds(g * GRP, GRP)], rbuf)
        pltpu.sync_copy(col2d.at[pl.ds(g * GRP, GRP)], cbuf)
        pltpu.sync_copy(w2d.at[pl.ds(g * GRP, GRP)], wbuf)
        gcps = [pltpu.async_copy(xs_sh.at[rbuf.at[0]], rows2.at[0], gsems[0]),
                None]
        scps = [None, None]
        for j in range(GRP):
            cur = j & 1
            if j + 1 < GRP:
                if scps[1 - cur] is not None:
                    scps[1 - cur].wait()
                gcps[1 - cur] = pltpu.async_copy(
                    xs_sh.at[rbuf.at[j + 1]], rows2.at[1 - cur], gsems[1 - cur])
            gcps[cur].wait()
            rows = rows2.at[cur]
            jvec = jnp.full((16,), j, jnp.int32)
            for v in range(CH // 2):
                ridx = 2 * v + ge8
                wv = plsc.load_gather(wbuf, [jvec, ridx])
                rv = plsc.load_gather(rows, [ridx, mod8])
                plsc.store_scatter(rows, [ridx, mod8], wv * rv)
            scps[cur] = pltpu.async_copy(
                rows, agg_sh.at[cbuf.at[j]], ssems[cur], add=True)
        scps[0].wait()
        scps[1].wait()
        return 0
    lax.fori_loop(0, trips, _outer, 0)

    plsc.subcore_barrier()
    pltpu.sync_copy(agg_sh.at[pl.ds(s * NPT, NPT)],
                    aggp.at[c, pl.ds(s * NPT, NPT)])


def _t1_body(degp_ref, x_ref, dinv_ref, xs_ref):
    deg = 1.0 + degp_ref[0, :] + degp_ref[1, :]
    dinv = lax.rsqrt(deg)
    dinv_ref[...] = dinv[:, None]
    xs_ref[...] = dinv[:, None] * x_ref[...]


def _t2_body(aggp_ref, xs_ref, dinv_ref, w_ref, b_ref, out_ref):
    t = (aggp_ref[0] + aggp_ref[1] + xs_ref[...]) * dinv_ref[...]
    out_ref[...] = (
        jnp.dot(t, w_ref[...], preferred_element_type=jnp.float32) + b_ref[...]
    )


_BT = 2048


def kernel(x, edge_index, edge_attr, W, b):
    row = edge_index[0].astype(jnp.int32)
    col = edge_index[1].astype(jnp.int32)
    w = edge_attr.astype(jnp.float32)
    ech = N_EDGES // CH
    row2d = row.reshape(ech, CH)
    col2d = col.reshape(ech, CH)
    w2d = w.reshape(ech, CH)
    x_p = jnp.pad(x, ((0, NP - N_NODES), (0, 0)))
    z8 = jnp.zeros((NP, 8), jnp.float32)

    degp = _deg_kernel(col2d, w2d)

    dinv, xs = pl.pallas_call(
        _t1_body,
        grid=(NP // _BT,),
        in_specs=[
            pl.BlockSpec((NC, _BT), lambda i: (0, i)),
            pl.BlockSpec((_BT, 8), lambda i: (i, 0)),
        ],
        out_specs=[
            pl.BlockSpec((_BT, 1), lambda i: (i, 0)),
            pl.BlockSpec((_BT, 8), lambda i: (i, 0)),
        ],
        out_shape=[
            jax.ShapeDtypeStruct((NP, 1), jnp.float32),
            jax.ShapeDtypeStruct((NP, 8), jnp.float32),
        ],
    )(degp, x_p)

    aggp = _agg_kernel(row2d, col2d, w2d, xs, z8)

    out = pl.pallas_call(
        _t2_body,
        grid=(NP // _BT,),
        in_specs=[
            pl.BlockSpec((NC, _BT, 8), lambda i: (0, i, 0)),
            pl.BlockSpec((_BT, 8), lambda i: (i, 0)),
            pl.BlockSpec((_BT, 1), lambda i: (i, 0)),
            pl.BlockSpec((8, 64), lambda i: (0, 0)),
            pl.BlockSpec((1, 64), lambda i: (0, 0)),
        ],
        out_specs=pl.BlockSpec((_BT, 64), lambda i: (i, 0)),
        out_shape=jax.ShapeDtypeStruct((NP, 64), jnp.float32),
    )(aggp, xs, dinv, W, b.reshape(1, 64))

    return out[:N_NODES]
